# baseline (speedup 1.0000x reference)
import functools

import jax
import jax.numpy as jnp
from jax import lax
from jax.experimental import pallas as pl
from jax.experimental.pallas import tpu as pltpu

_N_TOK = 8192
_K = 50176
_EMB = 256
_NEXP = 64

_K_BLK = 1024
_M_BLK = 2048

_NT = (((1,), (1,)), ((), ()))


def _fold_body(w_ref, keys_ref, b_ref, t_ref, wct_ref, bc_ref):
    k = pl.program_id(0)
    keys = keys_ref[...]
    norm = jnp.sqrt(jnp.sum(keys * keys, axis=1, keepdims=True))
    kn = keys / jnp.maximum(norm, 1e-12)
    scale = t_ref[0, 0] / jnp.sqrt(jnp.float32(_EMB))
    kn_s = kn * scale
    wct_ref[...] = jnp.dot(
        kn_s, w_ref[...], preferred_element_type=jnp.float32
    ).astype(jnp.bfloat16)

    @pl.when(k == 0)
    def _():
        bc_ref[...] = lax.dot_general(
            b_ref[...], kn_s, _NT, preferred_element_type=jnp.float32
        )


def _main_body(x_ref, wct_ref, bc_ref, o_ref):
    k = pl.program_id(1)
    part = lax.dot_general(
        x_ref[...].astype(jnp.bfloat16),
        wct_ref[...],
        _NT,
        preferred_element_type=jnp.float32,
    )

    @pl.when(k == 0)
    def _():
        o_ref[...] = part + bc_ref[...]

    @pl.when(k > 0)
    def _():
        o_ref[...] += part

    @pl.when(k == _K // _K_BLK - 1)
    def _():
        logit = o_ref[...]
        m = jnp.max(logit, axis=-1, keepdims=True)
        e = jnp.exp(logit - m)
        o_ref[...] = e / jnp.sum(e, axis=-1, keepdims=True)


def kernel(patch, layer_idx, threshold, W, b, keys, logit_temp):
    n = patch.shape[0]
    x = patch.reshape(n, _K)
    w_flat = W.reshape(_EMB, _K)
    b2 = b.reshape(1, _EMB)
    t2 = jnp.asarray(logit_temp, jnp.float32).reshape(1, 1)

    n_k = _K // _K_BLK

    wct, bc = pl.pallas_call(
        _fold_body,
        grid=(n_k,),
        in_specs=[
            pl.BlockSpec((_EMB, _K_BLK), lambda k: (0, k)),
            pl.BlockSpec((_NEXP, _EMB), lambda k: (0, 0)),
            pl.BlockSpec((1, _EMB), lambda k: (0, 0)),
            pl.BlockSpec((1, 1), lambda k: (0, 0)),
        ],
        out_specs=[
            pl.BlockSpec((_NEXP, _K_BLK), lambda k: (0, k)),
            pl.BlockSpec((1, _NEXP), lambda k: (0, 0)),
        ],
        out_shape=[
            jax.ShapeDtypeStruct((_NEXP, _K), jnp.bfloat16),
            jax.ShapeDtypeStruct((1, _NEXP), jnp.float32),
        ],
        compiler_params=pltpu.CompilerParams(
            dimension_semantics=("arbitrary",)
        ),
    )(w_flat, keys, b2, t2)

    n_m = n // _M_BLK
    out = pl.pallas_call(
        _main_body,
        grid=(n_m, n_k),
        in_specs=[
            pl.BlockSpec((_M_BLK, _K_BLK), lambda m, k: (m, k)),
            pl.BlockSpec((_NEXP, _K_BLK), lambda m, k: (0, k)),
            pl.BlockSpec((1, _NEXP), lambda m, k: (0, 0)),
        ],
        out_specs=pl.BlockSpec((_M_BLK, _NEXP), lambda m, k: (m, 0)),
        out_shape=jax.ShapeDtypeStruct((n, _NEXP), jnp.float32),
        compiler_params=pltpu.CompilerParams(
            dimension_semantics=("arbitrary", "arbitrary")
        ),
    )(x, wct, bc)
    return out


# speedup vs baseline: 3.7622x; 3.7622x over previous
import jax
import jax.numpy as jnp
from jax import lax
from jax.experimental import pallas as pl
from jax.experimental.pallas import tpu as pltpu

_N_TOK = 8192
_K = 50176
_EMB = 256
_NEXP = 64

_KF_BLK = 1024
_KM_BLK = 512

_NT = (((1,), (1,)), ((), ()))


def _fold_body(wt_ref, keys_ref, b_ref, t_ref, wct_ref, bc_ref):
    k = pl.program_id(0)
    keys = keys_ref[...]
    norm = jnp.sqrt(jnp.sum(keys * keys, axis=1, keepdims=True))
    kn = keys / jnp.maximum(norm, 1e-12)
    scale = t_ref[0, 0] / jnp.sqrt(jnp.float32(_EMB))
    kn_s = kn * scale
    wct_ref[...] = lax.dot_general(
        kn_s, wt_ref[...], _NT, preferred_element_type=jnp.float32
    ).astype(jnp.bfloat16)

    @pl.when(k == 0)
    def _():
        bc_ref[...] = lax.dot_general(
            kn_s, b_ref[...], _NT, preferred_element_type=jnp.float32
        )


def _main_body(xt_ref, wct_ref, bc_ref, o_ref):
    k = pl.program_id(0)
    part = jnp.dot(
        wct_ref[...],
        xt_ref[...].astype(jnp.bfloat16),
        preferred_element_type=jnp.float32,
    )

    @pl.when(k == 0)
    def _():
        o_ref[...] = part + bc_ref[...]

    @pl.when(k > 0)
    def _():
        o_ref[...] += part

    @pl.when(k == _K // _KM_BLK - 1)
    def _():
        logit = o_ref[...]
        m = jnp.max(logit, axis=0, keepdims=True)
        e = jnp.exp(logit - m)
        o_ref[...] = e / jnp.sum(e, axis=0, keepdims=True)


def kernel(patch, layer_idx, threshold, W, b, keys, logit_temp):
    n = patch.shape[0]
    xt = patch.reshape(n, _K).T
    wt = W.reshape(_EMB, _K).T
    b2 = b.reshape(1, _EMB)
    t2 = jnp.asarray(logit_temp, jnp.float32).reshape(1, 1)

    n_kf = _K // _KF_BLK
    wct, bc = pl.pallas_call(
        _fold_body,
        grid=(n_kf,),
        in_specs=[
            pl.BlockSpec((_KF_BLK, _EMB), lambda k: (k, 0)),
            pl.BlockSpec((_NEXP, _EMB), lambda k: (0, 0)),
            pl.BlockSpec((1, _EMB), lambda k: (0, 0)),
            pl.BlockSpec((1, 1), lambda k: (0, 0)),
        ],
        out_specs=[
            pl.BlockSpec((_NEXP, _KF_BLK), lambda k: (0, k)),
            pl.BlockSpec((_NEXP, 1), lambda k: (0, 0)),
        ],
        out_shape=[
            jax.ShapeDtypeStruct((_NEXP, _K), jnp.bfloat16),
            jax.ShapeDtypeStruct((_NEXP, 1), jnp.float32),
        ],
        compiler_params=pltpu.CompilerParams(
            dimension_semantics=("arbitrary",)
        ),
    )(wt, keys, b2, t2)

    n_km = _K // _KM_BLK
    out_t = pl.pallas_call(
        _main_body,
        grid=(n_km,),
        in_specs=[
            pl.BlockSpec((_KM_BLK, n), lambda k: (k, 0)),
            pl.BlockSpec((_NEXP, _KM_BLK), lambda k: (0, k)),
            pl.BlockSpec((_NEXP, 1), lambda k: (0, 0)),
        ],
        out_specs=pl.BlockSpec((_NEXP, n), lambda k: (0, 0)),
        out_shape=jax.ShapeDtypeStruct((_NEXP, n), jnp.float32),
        compiler_params=pltpu.CompilerParams(
            dimension_semantics=("arbitrary",)
        ),
    )(xt, wct, bc)
    return out_t.T
